# Initial kernel scaffold; baseline (speedup 1.0000x reference)
import jax
import jax.numpy as jnp
from jax.experimental import pallas as pl


def kernel(code, mean, basis, vec2texImg_index):
    raise NotImplementedError("write your pallas kernel here")


# speedup vs baseline: 1.0118x; 1.0118x over previous
import functools

import jax
import jax.numpy as jnp
from jax.experimental import pallas as pl
from jax.experimental.pallas import tpu as pltpu

_B = 16
_DIM = 50
_N_VEC = 1572864
_IMG_LEN = 1024 * 1024 * 3
_BLK = 8192


def _mm_body(code_ref, basis_ref, mean_ref, out_ref):
    prod = jax.lax.dot_general(
        basis_ref[...], code_ref[...],
        dimension_numbers=(((1,), (1,)), ((), ())),
        preferred_element_type=jnp.float32,
    )
    out_ref[...] = (prod + mean_ref[...]) * (1.0 / 255.0)


def _tex_code_T(code, mean, basis):
    n = basis.shape[0]
    grid = n // _BLK
    return pl.pallas_call(
        _mm_body,
        grid=(grid,),
        in_specs=[
            pl.BlockSpec((_B, _DIM), lambda i: (0, 0)),
            pl.BlockSpec((_BLK, _DIM), lambda i: (i, 0)),
            pl.BlockSpec((_BLK, 1), lambda i: (i, 0)),
        ],
        out_specs=pl.BlockSpec((_BLK, _B), lambda i: (i, 0)),
        out_shape=jax.ShapeDtypeStruct((n, _B), jnp.float32),
    )(code, basis, mean.reshape(n, 1))


def kernel(code, mean, basis, vec2texImg_index):
    tex_t = _tex_code_T(code, mean, basis)
    j = vec2texImg_index.astype(jnp.int32)
    c = j >> 20
    x = j & 1023
    y = (j >> 10) & 1023
    jt = (c << 20) | (x << 10) | y
    tex = jnp.zeros((_IMG_LEN, _B), dtype=jnp.float32).at[jt].add(tex_t)
    return tex.T.reshape(_B, 3, 1024, 1024)


# speedup vs baseline: 1.1683x; 1.1546x over previous
import functools

import jax
import jax.numpy as jnp
from jax import lax
from jax.experimental import pallas as pl
from jax.experimental.pallas import tpu as pltpu
from jax.experimental.pallas import tpu_sc as plsc

_B = 16
_DIM = 50
_N_VEC = 1572864
_IMG_LEN = 1024 * 1024 * 3
_BLK = 8192

_NC = 2
_NS = 16
_PER_TILE = _N_VEC // _NS
_HALF = _IMG_LEN // _NC
_R = 98304
_PASSES = _HALF // _R
_WSHARE = _R // _NS
_K = 5120
_C = 128
_ZROWS = 128
_CHUNK = 8192
_NCHUNK = _PER_TILE // _CHUNK


def _mm_body(code_ref, basis_ref, mean_ref, out_ref):
    prod = jax.lax.dot_general(
        basis_ref[...], code_ref[...],
        dimension_numbers=(((1,), (1,)), ((), ())),
        preferred_element_type=jnp.float32,
    )
    out_ref[...] = (prod + mean_ref[...]) * (1.0 / 255.0)

    @pl.when(pl.program_id(0) == pl.num_programs(0) - 1)
    def _():
        out_ref[...] = jnp.zeros_like(out_ref)


def _tex_code_T(code, mean, basis):
    n = basis.shape[0]
    grid = n // _BLK + 1
    last = n // _BLK - 1
    return pl.pallas_call(
        _mm_body,
        grid=(grid,),
        in_specs=[
            pl.BlockSpec((_B, _DIM), lambda i: (0, 0)),
            pl.BlockSpec((_BLK, _DIM), lambda i: (jnp.minimum(i, last), 0)),
            pl.BlockSpec((_BLK, 1), lambda i: (jnp.minimum(i, last), 0)),
        ],
        out_specs=pl.BlockSpec((_BLK, _B), lambda i: (i, 0)),
        out_shape=jax.ShapeDtypeStruct((n + _BLK, _B), jnp.float32),
    )(code, basis, mean.reshape(n, 1))


def _sc_body(idx_hbm, tex_hbm, out_hbm,
             ib_ref, jtl_ref, nl_ref, j2d_ref, n2d_ref, row_ref, zb_ref,
             acc_ref, sem):
    c = lax.axis_index("c")
    s = lax.axis_index("s")
    base_n = s * _PER_TILE

    zvec = jnp.zeros((16,), jnp.float32)

    def zb_init(i, _):
        zb_ref[i, :] = zvec
        return 0

    lax.fori_loop(0, _ZROWS, zb_init, 0)

    lane = lax.iota(jnp.int32, 16)
    sc_base = c * _HALF

    def pass_body(p, _):
        lo = sc_base + p * _R
        row0 = s * _WSHARE

        def zero_acc(i, _):
            pltpu.sync_copy(
                zb_ref, acc_ref.at[pl.ds(row0 + i * _ZROWS, _ZROWS), :])
            return 0

        lax.fori_loop(0, _WSHARE // _ZROWS, zero_acc, 0)
        plsc.subcore_barrier()

        def chunk_body(ch, cnt):
            pltpu.sync_copy(
                idx_hbm.at[pl.ds(base_n + ch * _CHUNK, _CHUNK)], ib_ref)

            def scan(i, cnt):
                v = ib_ref[pl.ds(i * 16, 16)]
                hi = v & jnp.int32(~0xFFFFF)
                x = v & jnp.int32(1023)
                y = (v >> 10) & jnp.int32(1023)
                d = (hi | (x << 10) | y) - lo
                m = d.astype(jnp.uint32) < jnp.uint32(_R)
                m = jnp.logical_and(m, cnt < _K)
                plsc.store_compressed(jtl_ref.at[pl.ds(cnt, 16)], d, mask=m)
                nvec = lane + (base_n + ch * _CHUNK + i * 16)
                plsc.store_compressed(nl_ref.at[pl.ds(cnt, 16)], nvec, mask=m)
                return cnt + jnp.sum(m.astype(jnp.int32))

            return lax.fori_loop(0, _CHUNK // 16, scan, cnt)

        cnt = lax.fori_loop(0, _NCHUNK, chunk_body, jnp.int32(0))

        for g in range(_C // 16):
            off = cnt + g * 16
            jtl_ref[pl.ds(off, 16)] = jnp.zeros((16,), jnp.int32)
            nl_ref[pl.ds(off, 16)] = lane + (_N_VEC + g * 16)

        def flush(j, _):
            base = j * _C
            for g in range(_C // 16):
                n2d_ref[0, pl.ds(g * 16, 16)] = nl_ref[pl.ds(base + g * 16, 16)]
                j2d_ref[0, pl.ds(g * 16, 16)] = jtl_ref[pl.ds(base + g * 16, 16)]
            pltpu.async_copy(tex_hbm.at[n2d_ref.at[0]], row_ref, sem).wait()
            pltpu.sync_copy(row_ref, acc_ref.at[j2d_ref.at[0]], add=True)
            return 0

        lax.fori_loop(0, (cnt + _C - 1) // _C, flush, 0)
        plsc.subcore_barrier()

        pltpu.sync_copy(
            acc_ref.at[pl.ds(row0, _WSHARE), :],
            out_hbm.at[pl.ds(lo + row0, _WSHARE), :])
        return 0

    lax.fori_loop(0, _PASSES, pass_body, 0)


def _sc_scatter(idx, tex_t):
    mesh = plsc.VectorSubcoreMesh(
        core_axis_name="c", subcore_axis_name="s",
        num_cores=_NC, num_subcores=_NS)
    return pl.kernel(
        _sc_body,
        out_type=jax.ShapeDtypeStruct((_IMG_LEN, _B), jnp.float32),
        mesh=mesh,
        compiler_params=pltpu.CompilerParams(
            needs_layout_passes=False, use_tc_tiling_on_sc=False),
        scratch_types=[
            pltpu.VMEM((_CHUNK,), jnp.int32),
            pltpu.VMEM((_K + _C,), jnp.int32),
            pltpu.VMEM((_K + _C,), jnp.int32),
            pltpu.VMEM((1, _C), jnp.int32),
            pltpu.VMEM((1, _C), jnp.int32),
            pltpu.VMEM((_C, _B), jnp.float32),
            pltpu.VMEM((_ZROWS, _B), jnp.float32),
            pltpu.VMEM_SHARED((_R, _B), jnp.float32),
            pltpu.SemaphoreType.DMA,
        ],
    )(idx, tex_t)


def kernel(code, mean, basis, vec2texImg_index):
    tex_t = _tex_code_T(code, mean, basis)
    out = _sc_scatter(vec2texImg_index.astype(jnp.int32), tex_t)
    return out.T.reshape(_B, 3, 1024, 1024)


# speedup vs baseline: 1.8065x; 1.5463x over previous
import functools

import jax
import jax.numpy as jnp
from jax import lax
from jax.experimental import pallas as pl
from jax.experimental.pallas import tpu as pltpu
from jax.experimental.pallas import tpu_sc as plsc

_B = 16
_DIM = 50
_N_VEC = 1572864
_IMG_LEN = 1024 * 1024 * 3
_BLK = 8192

_NC = 2
_NS = 16
_PER_TILE = _N_VEC // _NS
_HALF = _IMG_LEN // _NC
_R = 98304
_PASSES = _HALF // _R
_WSHARE = _R // _NS
_K = 5120
_C = 128
_ZROWS = 128
_CHUNK = 4096
_NCHUNK = _PER_TILE // _CHUNK


def _mm_body(code_ref, basis_ref, mean_ref, out_ref):
    prod = jax.lax.dot_general(
        basis_ref[...], code_ref[...],
        dimension_numbers=(((1,), (1,)), ((), ())),
        preferred_element_type=jnp.float32,
    )
    out_ref[...] = (prod + mean_ref[...].reshape(_BLK, 1)) * (1.0 / 255.0)

    @pl.when(pl.program_id(0) == pl.num_programs(0) - 1)
    def _():
        out_ref[...] = jnp.zeros_like(out_ref)


def _tex_code_T(code, mean, basis):
    n = basis.shape[0]
    grid = n // _BLK + 1
    last = n // _BLK - 1
    return pl.pallas_call(
        _mm_body,
        grid=(grid,),
        in_specs=[
            pl.BlockSpec((_B, _DIM), lambda i: (0, 0)),
            pl.BlockSpec((_BLK, _DIM), lambda i: (jnp.minimum(i, last), 0)),
            pl.BlockSpec((_BLK,), lambda i: (jnp.minimum(i, last),)),
        ],
        out_specs=pl.BlockSpec((_BLK, _B), lambda i: (i, 0)),
        out_shape=jax.ShapeDtypeStruct((n + _BLK, _B), jnp.float32),
    )(code, basis, mean)


def _sc_body(idx_hbm, tex_hbm, out_hbm,
             ib_ref, jtl_ref, nl_ref, j2d_ref, n2d_ref, row_ref, zb_ref,
             acc_ref, sem):
    c = lax.axis_index("c")
    s = lax.axis_index("s")
    base_n = s * _PER_TILE

    zvec = jnp.zeros((16,), jnp.float32)

    def zb_init(i, _):
        zb_ref[i, :] = zvec
        return 0

    lax.fori_loop(0, _ZROWS, zb_init, 0)

    lane = lax.iota(jnp.int32, 16)
    sc_base = c * _HALF

    def pass_body(p, _):
        lo = sc_base + p * _R
        row0 = s * _WSHARE

        def zero_acc(i, _):
            pltpu.sync_copy(
                zb_ref, acc_ref.at[pl.ds(row0 + i * _ZROWS, _ZROWS), :])
            return 0

        lax.fori_loop(0, _WSHARE // _ZROWS, zero_acc, 0)
        plsc.subcore_barrier()

        def chunk_body(ch, cnt):
            pltpu.sync_copy(
                idx_hbm.at[pl.ds(base_n + ch * _CHUNK, _CHUNK)], ib_ref)

            def scan(i, cnt):
                v = ib_ref[pl.ds(i * 16, 16)]
                hi = v & jnp.int32(~0xFFFFF)
                x = v & jnp.int32(1023)
                y = (v >> 10) & jnp.int32(1023)
                d = (hi | (x << 10) | y) - lo
                m = d.astype(jnp.uint32) < jnp.uint32(_R)
                plsc.store_compressed(jtl_ref.at[pl.ds(cnt, 16)], d, mask=m)
                nvec = lane + (base_n + ch * _CHUNK + i * 16)
                plsc.store_compressed(nl_ref.at[pl.ds(cnt, 16)], nvec, mask=m)
                return cnt + plsc.all_reduce_population_count(m)[0]

            cnt = plsc.parallel_loop(
                0, _CHUNK // 16, 1, unroll=8, carry=cnt)(scan)
            return jnp.minimum(cnt, jnp.int32(_K))

        cnt = lax.fori_loop(0, _NCHUNK, chunk_body, jnp.int32(0))

        for g in range(_C // 16):
            off = cnt + g * 16
            jtl_ref[pl.ds(off, 16)] = jnp.zeros((16,), jnp.int32)
            nl_ref[pl.ds(off, 16)] = lane + (_N_VEC + g * 16)

        def flush(j, _):
            base = j * _C
            for g in range(_C // 16):
                n2d_ref[0, pl.ds(g * 16, 16)] = nl_ref[pl.ds(base + g * 16, 16)]
                j2d_ref[0, pl.ds(g * 16, 16)] = jtl_ref[pl.ds(base + g * 16, 16)]
            pltpu.async_copy(tex_hbm.at[n2d_ref.at[0]], row_ref, sem).wait()
            pltpu.sync_copy(row_ref, acc_ref.at[j2d_ref.at[0]], add=True)
            return 0

        lax.fori_loop(0, (cnt + _C - 1) // _C, flush, 0)
        plsc.subcore_barrier()

        pltpu.sync_copy(
            acc_ref.at[pl.ds(row0, _WSHARE), :],
            out_hbm.at[pl.ds(lo + row0, _WSHARE), :])
        return 0

    lax.fori_loop(0, _PASSES, pass_body, 0)


def _sc_scatter(idx, tex_t):
    mesh = plsc.VectorSubcoreMesh(
        core_axis_name="c", subcore_axis_name="s",
        num_cores=_NC, num_subcores=_NS)
    return pl.kernel(
        _sc_body,
        out_type=jax.ShapeDtypeStruct((_IMG_LEN, _B), jnp.float32),
        mesh=mesh,
        compiler_params=pltpu.CompilerParams(
            needs_layout_passes=False, use_tc_tiling_on_sc=False),
        scratch_types=[
            pltpu.VMEM((_CHUNK,), jnp.int32),
            pltpu.VMEM((_K + _CHUNK + _C,), jnp.int32),
            pltpu.VMEM((_K + _CHUNK + _C,), jnp.int32),
            pltpu.VMEM((1, _C), jnp.int32),
            pltpu.VMEM((1, _C), jnp.int32),
            pltpu.VMEM((_C, _B), jnp.float32),
            pltpu.VMEM((_ZROWS, _B), jnp.float32),
            pltpu.VMEM_SHARED((_R, _B), jnp.float32),
            pltpu.SemaphoreType.DMA,
        ],
    )(idx, tex_t)


def kernel(code, mean, basis, vec2texImg_index):
    tex_t = _tex_code_T(code, mean, basis)
    out = _sc_scatter(vec2texImg_index.astype(jnp.int32), tex_t)
    return out.T.reshape(_B, 3, 1024, 1024)


# speedup vs baseline: 1.8914x; 1.0470x over previous
import functools

import jax
import jax.numpy as jnp
from jax import lax
from jax.experimental import pallas as pl
from jax.experimental.pallas import tpu as pltpu
from jax.experimental.pallas import tpu_sc as plsc

_B = 16
_DIM = 50
_N_VEC = 1572864
_IMG_LEN = 1024 * 1024 * 3
_BLK = 8192

_NC = 2
_NS = 16
_PER_TILE = _N_VEC // _NS
_HALF = _IMG_LEN // _NC
_R = 98304
_PASSES = _HALF // _R
_WSHARE = _R // _NS
_K = 5120
_C = 128
_ZROWS = 128
_CHUNK = 4096
_NCHUNK = _PER_TILE // _CHUNK


def _mm_body(code_ref, basis_ref, mean_ref, out_ref):
    prod = jax.lax.dot_general(
        basis_ref[...], code_ref[...],
        dimension_numbers=(((1,), (1,)), ((), ())),
        preferred_element_type=jnp.float32,
    )
    out_ref[...] = (prod + mean_ref[...].reshape(_BLK, 1)) * (1.0 / 255.0)

    @pl.when(pl.program_id(0) == pl.num_programs(0) - 1)
    def _():
        out_ref[...] = jnp.zeros_like(out_ref)


def _tex_code_T(code, mean, basis):
    n = basis.shape[0]
    grid = n // _BLK + 1
    last = n // _BLK - 1
    return pl.pallas_call(
        _mm_body,
        grid=(grid,),
        in_specs=[
            pl.BlockSpec((_B, _DIM), lambda i: (0, 0)),
            pl.BlockSpec((_BLK, _DIM), lambda i: (jnp.minimum(i, last), 0)),
            pl.BlockSpec((_BLK,), lambda i: (jnp.minimum(i, last),)),
        ],
        out_specs=pl.BlockSpec((_BLK, _B), lambda i: (i, 0)),
        out_shape=jax.ShapeDtypeStruct((n + _BLK, _B), jnp.float32),
    )(code, basis, mean)


def _sc_body(idx_hbm, tex_hbm, out_hbm,
             ib_ref, jtl_ref, nl_ref, j2d_ref, n2d_ref, row_ref, zb_ref,
             acc_ref, sem, sem2):
    c = lax.axis_index("c")
    s = lax.axis_index("s")
    base_n = s * _PER_TILE

    zvec = jnp.zeros((16,), jnp.float32)

    def zb_init(i, _):
        zb_ref[i, :] = zvec
        return 0

    lax.fori_loop(0, _ZROWS, zb_init, 0)

    lane = lax.iota(jnp.int32, 16)
    sc_base = c * _HALF

    def pass_body(p, _):
        lo = sc_base + p * _R
        row0 = s * _WSHARE

        def zero_acc(i, _):
            pltpu.sync_copy(
                zb_ref, acc_ref.at[pl.ds(row0 + i * _ZROWS, _ZROWS), :])
            return 0

        lax.fori_loop(0, _WSHARE // _ZROWS, zero_acc, 0)
        plsc.subcore_barrier()

        def chunk_body(ch, cnt):
            pltpu.sync_copy(
                idx_hbm.at[pl.ds(base_n + ch * _CHUNK, _CHUNK)], ib_ref)

            def scan(i, cnt):
                v = ib_ref[pl.ds(i * 16, 16)]
                hi = v & jnp.int32(~0xFFFFF)
                x = v & jnp.int32(1023)
                y = (v >> 10) & jnp.int32(1023)
                d = (hi | (x << 10) | y) - lo
                m = d.astype(jnp.uint32) < jnp.uint32(_R)
                plsc.store_compressed(jtl_ref.at[pl.ds(cnt, 16)], d, mask=m)
                nvec = lane + (base_n + ch * _CHUNK + i * 16)
                plsc.store_compressed(nl_ref.at[pl.ds(cnt, 16)], nvec, mask=m)
                return cnt + plsc.all_reduce_population_count(m)[0]

            cnt = plsc.parallel_loop(
                0, _CHUNK // 16, 1, unroll=8, carry=cnt)(scan)
            return jnp.minimum(cnt, jnp.int32(_K))

        cnt = lax.fori_loop(0, _NCHUNK, chunk_body, jnp.int32(0))

        for g in range(_C // 16):
            off = cnt + g * 16
            jtl_ref[pl.ds(off, 16)] = jnp.zeros((16,), jnp.int32)
            nl_ref[pl.ds(off, 16)] = lane + (_N_VEC + g * 16)

        nf = (cnt + _C - 1) // _C

        def stage(j, buf):
            base = j * _C
            for g in range(_C // 16):
                sl = pl.ds(g * 16, 16)
                n2d_ref[buf, sl] = nl_ref[pl.ds(base + g * 16, 16)]
                j2d_ref[buf, sl] = jtl_ref[pl.ds(base + g * 16, 16)]

        def gather_start(buf, s):
            pltpu.async_copy(tex_hbm.at[n2d_ref.at[buf]], row_ref.at[buf], s)

        def gather_wait(buf, s):
            pltpu.make_async_copy(
                tex_hbm.at[n2d_ref.at[buf]], row_ref.at[buf], s).wait()

        def scatter(buf):
            pltpu.sync_copy(
                row_ref.at[buf], acc_ref.at[j2d_ref.at[buf]], add=True)

        @pl.when(nf > 0)
        def _():
            stage(0, 0)
            gather_start(0, sem)

        def flush2(k, _):
            j0 = k * 2

            @pl.when(j0 + 1 < nf)
            def _():
                stage(j0 + 1, 1)
                gather_start(1, sem2)

            gather_wait(0, sem)
            scatter(0)

            @pl.when(j0 + 1 < nf)
            def _():
                @pl.when(j0 + 2 < nf)
                def _():
                    stage(j0 + 2, 0)
                    gather_start(0, sem)

                gather_wait(1, sem2)
                scatter(1)

            return 0

        lax.fori_loop(0, (nf + 1) // 2, flush2, 0)
        plsc.subcore_barrier()

        pltpu.sync_copy(
            acc_ref.at[pl.ds(row0, _WSHARE), :],
            out_hbm.at[pl.ds(lo + row0, _WSHARE), :])
        return 0

    lax.fori_loop(0, _PASSES, pass_body, 0)


def _sc_scatter(idx, tex_t):
    mesh = plsc.VectorSubcoreMesh(
        core_axis_name="c", subcore_axis_name="s",
        num_cores=_NC, num_subcores=_NS)
    return pl.kernel(
        _sc_body,
        out_type=jax.ShapeDtypeStruct((_IMG_LEN, _B), jnp.float32),
        mesh=mesh,
        compiler_params=pltpu.CompilerParams(
            needs_layout_passes=False, use_tc_tiling_on_sc=False),
        scratch_types=[
            pltpu.VMEM((_CHUNK,), jnp.int32),
            pltpu.VMEM((_K + _CHUNK + _C,), jnp.int32),
            pltpu.VMEM((_K + _CHUNK + _C,), jnp.int32),
            pltpu.VMEM((2, _C), jnp.int32),
            pltpu.VMEM((2, _C), jnp.int32),
            pltpu.VMEM((2, _C, _B), jnp.float32),
            pltpu.VMEM((_ZROWS, _B), jnp.float32),
            pltpu.VMEM_SHARED((_R, _B), jnp.float32),
            pltpu.SemaphoreType.DMA,
            pltpu.SemaphoreType.DMA,
        ],
    )(idx, tex_t)


def kernel(code, mean, basis, vec2texImg_index):
    tex_t = _tex_code_T(code, mean, basis)
    out = _sc_scatter(vec2texImg_index.astype(jnp.int32), tex_t)
    return out.T.reshape(_B, 3, 1024, 1024)


# speedup vs baseline: 2.2053x; 1.1659x over previous
import functools

import jax
import jax.numpy as jnp
from jax import lax
from jax.experimental import pallas as pl
from jax.experimental.pallas import tpu as pltpu
from jax.experimental.pallas import tpu_sc as plsc

_B = 16
_DIM = 50
_N_VEC = 1572864
_IMG_LEN = 1024 * 1024 * 3
_BLK = 8192

_NC = 2
_NS = 16
_PER_TILE = _N_VEC // _NS
_HALF = _IMG_LEN // _NC
_R = 98304
_PASSES = _HALF // _R
_WSHARE = _R // _NS
_K = 5120
_C = 128
_ZROWS = 128
_CHUNK = 4096
_NCHUNK = _PER_TILE // _CHUNK


def _mm_body(code_ref, basis_ref, mean_ref, out_ref):
    prod = jax.lax.dot_general(
        basis_ref[...], code_ref[...],
        dimension_numbers=(((0,), (1,)), ((), ())),
        preferred_element_type=jnp.float32,
    )
    out_ref[...] = (prod + mean_ref[...].reshape(_BLK, 1)) * (1.0 / 255.0)

    @pl.when(pl.program_id(0) == pl.num_programs(0) - 1)
    def _():
        out_ref[...] = jnp.zeros_like(out_ref)


def _tex_code_T(code, mean, basis):
    n = basis.shape[0]
    grid = n // _BLK + 1
    last = n // _BLK - 1
    return pl.pallas_call(
        _mm_body,
        grid=(grid,),
        in_specs=[
            pl.BlockSpec((_B, _DIM), lambda i: (0, 0)),
            pl.BlockSpec((_DIM, _BLK), lambda i: (0, jnp.minimum(i, last))),
            pl.BlockSpec((_BLK,), lambda i: (jnp.minimum(i, last),)),
        ],
        out_specs=pl.BlockSpec((_BLK, _B), lambda i: (i, 0)),
        out_shape=jax.ShapeDtypeStruct((n + _BLK, _B), jnp.float32),
    )(code, jnp.swapaxes(basis, 0, 1), mean)


def _sc_body(idx_hbm, tex_hbm, out_hbm,
             ib_ref, jtl_ref, nl_ref, j2d_ref, n2d_ref, row_ref, zb_ref,
             acc_ref, sem, sem2):
    c = lax.axis_index("c")
    s = lax.axis_index("s")
    base_n = s * _PER_TILE

    zvec = jnp.zeros((16,), jnp.float32)

    def zb_init(i, _):
        zb_ref[i, :] = zvec
        return 0

    lax.fori_loop(0, _ZROWS, zb_init, 0)

    lane = lax.iota(jnp.int32, 16)
    sc_base = c * _HALF

    def pass_body(p, _):
        lo = sc_base + p * _R
        row0 = s * _WSHARE

        def zero_acc(i, _):
            pltpu.sync_copy(
                zb_ref, acc_ref.at[pl.ds(row0 + i * _ZROWS, _ZROWS), :])
            return 0

        lax.fori_loop(0, _WSHARE // _ZROWS, zero_acc, 0)
        plsc.subcore_barrier()

        def chunk_body(ch, cnt):
            pltpu.sync_copy(
                idx_hbm.at[pl.ds(base_n + ch * _CHUNK, _CHUNK)], ib_ref)

            def scan(i, cnt):
                v = ib_ref[pl.ds(i * 16, 16)]
                hi = v & jnp.int32(~0xFFFFF)
                x = v & jnp.int32(1023)
                y = (v >> 10) & jnp.int32(1023)
                d = (hi | (x << 10) | y) - lo
                m = d.astype(jnp.uint32) < jnp.uint32(_R)
                plsc.store_compressed(jtl_ref.at[pl.ds(cnt, 16)], d, mask=m)
                nvec = lane + (base_n + ch * _CHUNK + i * 16)
                plsc.store_compressed(nl_ref.at[pl.ds(cnt, 16)], nvec, mask=m)
                return cnt + plsc.all_reduce_population_count(m)[0]

            cnt = plsc.parallel_loop(
                0, _CHUNK // 16, 1, unroll=8, carry=cnt)(scan)
            return jnp.minimum(cnt, jnp.int32(_K))

        cnt = lax.fori_loop(0, _NCHUNK, chunk_body, jnp.int32(0))

        for g in range(_C // 16):
            off = cnt + g * 16
            jtl_ref[pl.ds(off, 16)] = jnp.zeros((16,), jnp.int32)
            nl_ref[pl.ds(off, 16)] = lane + (_N_VEC + g * 16)

        nf = (cnt + _C - 1) // _C

        def stage(j, buf):
            base = j * _C
            for g in range(_C // 16):
                sl = pl.ds(g * 16, 16)
                n2d_ref[buf, sl] = nl_ref[pl.ds(base + g * 16, 16)]
                j2d_ref[buf, sl] = jtl_ref[pl.ds(base + g * 16, 16)]

        def gather_start(buf, s):
            pltpu.async_copy(tex_hbm.at[n2d_ref.at[buf]], row_ref.at[buf], s)

        def gather_wait(buf, s):
            pltpu.make_async_copy(
                tex_hbm.at[n2d_ref.at[buf]], row_ref.at[buf], s).wait()

        def scatter(buf):
            pltpu.sync_copy(
                row_ref.at[buf], acc_ref.at[j2d_ref.at[buf]], add=True)

        @pl.when(nf > 0)
        def _():
            stage(0, 0)
            gather_start(0, sem)

        def flush2(k, _):
            j0 = k * 2

            @pl.when(j0 + 1 < nf)
            def _():
                stage(j0 + 1, 1)
                gather_start(1, sem2)

            gather_wait(0, sem)
            scatter(0)

            @pl.when(j0 + 1 < nf)
            def _():
                @pl.when(j0 + 2 < nf)
                def _():
                    stage(j0 + 2, 0)
                    gather_start(0, sem)

                gather_wait(1, sem2)
                scatter(1)

            return 0

        lax.fori_loop(0, (nf + 1) // 2, flush2, 0)
        plsc.subcore_barrier()

        pltpu.sync_copy(
            acc_ref.at[pl.ds(row0, _WSHARE), :],
            out_hbm.at[pl.ds(lo + row0, _WSHARE), :])
        return 0

    lax.fori_loop(0, _PASSES, pass_body, 0)


def _sc_scatter(idx, tex_t):
    mesh = plsc.VectorSubcoreMesh(
        core_axis_name="c", subcore_axis_name="s",
        num_cores=_NC, num_subcores=_NS)
    return pl.kernel(
        _sc_body,
        out_type=jax.ShapeDtypeStruct((_IMG_LEN, _B), jnp.float32),
        mesh=mesh,
        compiler_params=pltpu.CompilerParams(
            needs_layout_passes=False, use_tc_tiling_on_sc=False),
        scratch_types=[
            pltpu.VMEM((_CHUNK,), jnp.int32),
            pltpu.VMEM((_K + _CHUNK + _C,), jnp.int32),
            pltpu.VMEM((_K + _CHUNK + _C,), jnp.int32),
            pltpu.VMEM((2, _C), jnp.int32),
            pltpu.VMEM((2, _C), jnp.int32),
            pltpu.VMEM((2, _C, _B), jnp.float32),
            pltpu.VMEM((_ZROWS, _B), jnp.float32),
            pltpu.VMEM_SHARED((_R, _B), jnp.float32),
            pltpu.SemaphoreType.DMA,
            pltpu.SemaphoreType.DMA,
        ],
    )(idx, tex_t)


def kernel(code, mean, basis, vec2texImg_index):
    tex_t = _tex_code_T(code, mean, basis)
    out = _sc_scatter(vec2texImg_index.astype(jnp.int32), tex_t)
    return out.T.reshape(_B, 3, 1024, 1024)


# speedup vs baseline: 2.3216x; 1.0527x over previous
import functools

import jax
import jax.numpy as jnp
from jax import lax
from jax.experimental import pallas as pl
from jax.experimental.pallas import tpu as pltpu
from jax.experimental.pallas import tpu_sc as plsc

_B = 16
_DIM = 50
_N_VEC = 1572864
_IMG_LEN = 1024 * 1024 * 3
_BLK = 8192

_NC = 2
_NS = 16
_PER_TILE = _N_VEC // _NS
_HALF = _IMG_LEN // _NC
_R = 98304
_PASSES = _HALF // _R
_WSHARE = _R // _NS
_K = 5120
_C = 128
_ZROWS = 128
_CHUNK = 2048
_NCHUNK = _PER_TILE // _CHUNK


def _mm_body(code_ref, basis_ref, mean_ref, out_ref):
    prod = jax.lax.dot_general(
        basis_ref[...], code_ref[...],
        dimension_numbers=(((0,), (1,)), ((), ())),
        preferred_element_type=jnp.float32,
    )
    out_ref[...] = (prod + mean_ref[...].reshape(_BLK, 1)) * (1.0 / 255.0)

    @pl.when(pl.program_id(0) == pl.num_programs(0) - 1)
    def _():
        out_ref[...] = jnp.zeros_like(out_ref)


def _tex_code_T(code, mean, basis):
    n = basis.shape[0]
    grid = n // _BLK + 1
    last = n // _BLK - 1
    return pl.pallas_call(
        _mm_body,
        grid=(grid,),
        in_specs=[
            pl.BlockSpec((_B, _DIM), lambda i: (0, 0)),
            pl.BlockSpec((_DIM, _BLK), lambda i: (0, jnp.minimum(i, last))),
            pl.BlockSpec((_BLK,), lambda i: (jnp.minimum(i, last),)),
        ],
        out_specs=pl.BlockSpec((_BLK, _B), lambda i: (i, 0)),
        out_shape=jax.ShapeDtypeStruct((n + _BLK, _B), jnp.float32),
    )(code, jnp.swapaxes(basis, 0, 1), mean)


def _sc_body(idx_hbm, tex_hbm, out_hbm,
             ib_ref, jtl_ref, nl_ref, j2d_ref, n2d_ref, row_ref, zb_ref,
             acc_ref, sem, sem2, semi, semi2):
    c = lax.axis_index("c")
    s = lax.axis_index("s")
    base_n = s * _PER_TILE

    zvec = jnp.zeros((16,), jnp.float32)

    def zb_init(i, _):
        zb_ref[i, :] = zvec
        return 0

    lax.fori_loop(0, _ZROWS, zb_init, 0)

    lane = lax.iota(jnp.int32, 16)
    sc_base = c * _HALF

    def pass_body(p, _):
        lo = sc_base + p * _R
        row0 = s * _WSHARE

        def zero_acc(i, _):
            pltpu.sync_copy(
                zb_ref, acc_ref.at[pl.ds(row0 + i * _ZROWS, _ZROWS), :])
            return 0

        lax.fori_loop(0, _WSHARE // _ZROWS, zero_acc, 0)
        plsc.subcore_barrier()

        def load_start(ch, buf, s):
            pltpu.async_copy(
                idx_hbm.at[pl.ds(base_n + ch * _CHUNK, _CHUNK)],
                ib_ref.at[buf], s)

        def load_wait(ch, buf, s):
            pltpu.make_async_copy(
                idx_hbm.at[pl.ds(base_n + ch * _CHUNK, _CHUNK)],
                ib_ref.at[buf], s).wait()

        def scan_chunk(ch, buf, cnt):
            def scan(i, cnt):
                v = ib_ref[buf, pl.ds(i * 16, 16)]
                hi = v & jnp.int32(~0xFFFFF)
                x = v & jnp.int32(1023)
                y = (v >> 10) & jnp.int32(1023)
                d = (hi | (x << 10) | y) - lo
                m = d.astype(jnp.uint32) < jnp.uint32(_R)
                plsc.store_compressed(jtl_ref.at[pl.ds(cnt, 16)], d, mask=m)
                nvec = lane + (base_n + ch * _CHUNK + i * 16)
                plsc.store_compressed(nl_ref.at[pl.ds(cnt, 16)], nvec, mask=m)
                return cnt + plsc.all_reduce_population_count(m)[0]

            cnt = plsc.parallel_loop(
                0, _CHUNK // 16, 1, unroll=8, carry=cnt)(scan)
            return jnp.minimum(cnt, jnp.int32(_K))

        load_start(0, 0, semi)

        def chunk_pair(k, cnt):
            ch0 = 2 * k
            load_start(ch0 + 1, 1, semi2)
            load_wait(ch0, 0, semi)
            cnt = scan_chunk(ch0, 0, cnt)

            @pl.when(ch0 + 2 < _NCHUNK)
            def _():
                load_start(ch0 + 2, 0, semi)

            load_wait(ch0 + 1, 1, semi2)
            return scan_chunk(ch0 + 1, 1, cnt)

        cnt = lax.fori_loop(0, _NCHUNK // 2, chunk_pair, jnp.int32(0))

        for g in range(_C // 16):
            off = cnt + g * 16
            jtl_ref[pl.ds(off, 16)] = jnp.zeros((16,), jnp.int32)
            nl_ref[pl.ds(off, 16)] = lane + (_N_VEC + g * 16)

        nf = (cnt + _C - 1) // _C

        def stage(j, buf):
            base = j * _C
            for g in range(_C // 16):
                sl = pl.ds(g * 16, 16)
                n2d_ref[buf, sl] = nl_ref[pl.ds(base + g * 16, 16)]
                j2d_ref[buf, sl] = jtl_ref[pl.ds(base + g * 16, 16)]

        def gather_start(buf, s):
            pltpu.async_copy(tex_hbm.at[n2d_ref.at[buf]], row_ref.at[buf], s)

        def gather_wait(buf, s):
            pltpu.make_async_copy(
                tex_hbm.at[n2d_ref.at[buf]], row_ref.at[buf], s).wait()

        def scatter(buf):
            pltpu.sync_copy(
                row_ref.at[buf], acc_ref.at[j2d_ref.at[buf]], add=True)

        @pl.when(nf > 0)
        def _():
            stage(0, 0)
            gather_start(0, sem)

        def flush2(k, _):
            j0 = k * 2

            @pl.when(j0 + 1 < nf)
            def _():
                stage(j0 + 1, 1)
                gather_start(1, sem2)

            gather_wait(0, sem)
            scatter(0)

            @pl.when(j0 + 1 < nf)
            def _():
                @pl.when(j0 + 2 < nf)
                def _():
                    stage(j0 + 2, 0)
                    gather_start(0, sem)

                gather_wait(1, sem2)
                scatter(1)

            return 0

        lax.fori_loop(0, (nf + 1) // 2, flush2, 0)
        plsc.subcore_barrier()

        pltpu.sync_copy(
            acc_ref.at[pl.ds(row0, _WSHARE), :],
            out_hbm.at[pl.ds(lo + row0, _WSHARE), :])
        return 0

    lax.fori_loop(0, _PASSES, pass_body, 0)


def _sc_scatter(idx, tex_t):
    mesh = plsc.VectorSubcoreMesh(
        core_axis_name="c", subcore_axis_name="s",
        num_cores=_NC, num_subcores=_NS)
    return pl.kernel(
        _sc_body,
        out_type=jax.ShapeDtypeStruct((_IMG_LEN, _B), jnp.float32),
        mesh=mesh,
        compiler_params=pltpu.CompilerParams(
            needs_layout_passes=False, use_tc_tiling_on_sc=False),
        scratch_types=[
            pltpu.VMEM((2, _CHUNK), jnp.int32),
            pltpu.VMEM((_K + _CHUNK + _C,), jnp.int32),
            pltpu.VMEM((_K + _CHUNK + _C,), jnp.int32),
            pltpu.VMEM((2, _C), jnp.int32),
            pltpu.VMEM((2, _C), jnp.int32),
            pltpu.VMEM((2, _C, _B), jnp.float32),
            pltpu.VMEM((_ZROWS, _B), jnp.float32),
            pltpu.VMEM_SHARED((_R, _B), jnp.float32),
            pltpu.SemaphoreType.DMA,
            pltpu.SemaphoreType.DMA,
            pltpu.SemaphoreType.DMA,
            pltpu.SemaphoreType.DMA,
        ],
    )(idx, tex_t)


def kernel(code, mean, basis, vec2texImg_index):
    tex_t = _tex_code_T(code, mean, basis)
    out = _sc_scatter(vec2texImg_index.astype(jnp.int32), tex_t)
    return out.T.reshape(_B, 3, 1024, 1024)
